# Initial kernel scaffold; baseline (speedup 1.0000x reference)
import jax
import jax.numpy as jnp
from jax.experimental import pallas as pl


def kernel(x, W_dense, b_dense, W_edge1, b_edge1, W_root1, bias1, bn1_g, bn1_b, W_edge2, b_edge2, W_root2, bias2, bn2_g, bn2_b):
    raise NotImplementedError("write your pallas kernel here")


# speedup vs baseline: 1.5276x; 1.5276x over previous
import jax
import jax.numpy as jnp
from jax.experimental import pallas as pl
from jax.experimental.pallas import tpu as pltpu

B = 1024
NH = 128
LD = 128
H0 = 16
H1 = 16
NF = 3
ALPHA = 0.2
N = B * NH

G = 8
NB = B // G
RB = 8192

_HI = jax.lax.Precision.HIGHEST
_BF = jnp.bfloat16


def _leaky(x):
    return jnp.where(x >= 0, x, ALPHA * x)


def _dot_bf(a, b):
    return jnp.dot(a.astype(_BF), b.astype(_BF),
                   preferred_element_type=jnp.float32)


def _split3(a):
    p1 = a.astype(_BF).astype(jnp.float32)
    r = a - p1
    p2 = r.astype(_BF).astype(jnp.float32)
    p3 = (r - p2).astype(_BF).astype(jnp.float32)
    return p1, p2, p3


def _sel_dot(sel01, b):
    p1, p2, p3 = _split3(b)
    return _dot_bf(sel01, p1) + _dot_bf(sel01, p2) + _dot_bf(sel01, p3)


def _red_dot(a, red01):
    p1, p2, p3 = _split3(a)
    return _dot_bf(p1, red01) + _dot_bf(p2, red01) + _dot_bf(p3, red01)


def _dense_kernel(x_ref, w_ref, b_ref, o_ref):
    o_ref[...] = _leaky(_dot_bf(x_ref[...], w_ref[...]) + b_ref[...])


def _graph_layer(h_g, we, be, wr, bias, d_out):
    C = 16 * d_out
    hb = h_g.astype(_BF)
    dots = jax.lax.dot_general(hb, hb, (((1,), (1,)), ((), ())),
                               preferred_element_type=jnp.float32)
    sq = jnp.sum(h_g * h_g, axis=1, keepdims=True)
    d2 = sq + jnp.transpose(sq) - 2.0 * dots
    iota_n = jax.lax.broadcasted_iota(jnp.int32, (NH, NH), 0)
    iota_m = jax.lax.broadcasted_iota(jnp.int32, (NH, NH), 1)
    d2 = d2 + jnp.where(iota_n == iota_m, 1e10, 0.0)
    rowmin = jnp.min(d2, axis=1, keepdims=True)
    iota_f = iota_m.astype(jnp.float32)
    cand = jnp.where(d2 <= rowmin, iota_f, jnp.float32(NH))
    nbr = jnp.min(cand, axis=1, keepdims=True)
    sel = jnp.where(cand == nbr, 1.0, 0.0)
    xs = _sel_dot(sel, h_g)
    ea = xs - h_g
    theta = _dot_bf(ea, we) + be
    ei = jax.lax.broadcasted_iota(jnp.int32, (16, C), 0)
    ec = jax.lax.broadcasted_iota(jnp.int32, (16, C), 1)
    e1 = jnp.where(ec // d_out == ei, 1.0, 0.0)
    xse = _dot_bf(xs, e1)
    thb = theta.astype(_BF).astype(jnp.float32)
    prod = xse * thb
    ri = jax.lax.broadcasted_iota(jnp.int32, (C, 16), 0)
    rc = jax.lax.broadcasted_iota(jnp.int32, (C, 16), 1)
    red = jnp.where(ri % d_out == rc, 1.0, 0.0)
    msg = _red_dot(prod, red)
    return _dot_bf(h_g, wr) + msg + bias


def _l1_kernel(h_ref, we_ref, be_ref, wr_ref, b_ref,
               y_ref, ps_ref, pq_ref):
    acc_s = jnp.zeros((1, 16), jnp.float32)
    acc_q = jnp.zeros((1, 16), jnp.float32)
    we = we_ref[...]
    be = be_ref[...]
    wr = wr_ref[...]
    bias = b_ref[...]
    for g in range(G):
        h_g = h_ref[g * NH:(g + 1) * NH, :]
        y_g = _graph_layer(h_g, we, be, wr, bias, 16)
        y_ref[g * NH:(g + 1) * NH, :] = y_g
        acc_s = acc_s + jnp.sum(y_g, axis=0, keepdims=True)
        acc_q = acc_q + jnp.sum(y_g * y_g, axis=0, keepdims=True)
    ps_ref[...] = acc_s.reshape(1, 1, 16)
    pq_ref[...] = acc_q.reshape(1, 1, 16)


def _l2_kernel(y1_ref, ps1_ref, pq1_ref, g1_ref, b1_ref,
               we_ref, be_ref, wr_ref, bias_ref,
               y2_ref, ps_ref, pq_ref):
    s = jnp.sum(ps1_ref[...], axis=(0, 1))
    q = jnp.sum(pq1_ref[...], axis=(0, 1))
    mean = (s / N).reshape(1, 16)
    var = (q / N).reshape(1, 16) - mean * mean
    denom = jnp.sqrt(var + 1e-5)
    gam = g1_ref[...]
    bet = b1_ref[...]
    we = we_ref[...]
    be = be_ref[...]
    wr = wr_ref[...]
    bias = bias_ref[...]
    acc_s = jnp.zeros((1, 16), jnp.float32)
    acc_q = jnp.zeros((1, 16), jnp.float32)
    for g in range(G):
        y1_g = y1_ref[g * NH:(g + 1) * NH, :]
        h_g = _leaky((y1_g - mean) / denom * gam + bet)
        y_g = _graph_layer(h_g, we, be, wr, bias, NF)
        y2_ref[g * NH:(g + 1) * NH, :] = y_g
        acc_s = acc_s + jnp.sum(y_g, axis=0, keepdims=True)
        acc_q = acc_q + jnp.sum(y_g * y_g, axis=0, keepdims=True)
    ps_ref[...] = acc_s.reshape(1, 1, 16)
    pq_ref[...] = acc_q.reshape(1, 1, 16)


def _bn_kernel(y2_ref, ps_ref, pq_ref, g_ref, b_ref, o_ref):
    s = jnp.sum(ps_ref[...], axis=(0, 1))
    q = jnp.sum(pq_ref[...], axis=(0, 1))
    mean = (s / N).reshape(1, 16)
    var = (q / N).reshape(1, 16) - mean * mean
    denom = jnp.sqrt(var + 1e-5)
    o_ref[...] = _leaky((y2_ref[...] - mean) / denom * g_ref[...] + b_ref[...])


def kernel(x, W_dense, b_dense, W_edge1, b_edge1, W_root1, bias1, bn1_g, bn1_b,
           W_edge2, b_edge2, W_root2, bias2, bn2_g, bn2_b):
    f32 = jnp.float32
    we1 = W_edge1
    be1 = b_edge1.reshape(1, H0 * H1)
    wr1 = W_root1
    bs1 = bias1.reshape(1, H1)
    g1 = bn1_g.reshape(1, H1)
    b1 = bn1_b.reshape(1, H1)
    we2 = W_edge2
    be2 = b_edge2.reshape(1, H1 * NF)
    wr2 = jnp.pad(W_root2, ((0, 0), (0, 16 - NF)))
    bs2 = jnp.pad(bias2.reshape(1, NF), ((0, 0), (0, 16 - NF)))
    g2 = jnp.pad(bn2_g.reshape(1, NF), ((0, 0), (0, 16 - NF)))
    b2 = jnp.pad(bn2_b.reshape(1, NF), ((0, 0), (0, 16 - NF)))
    bd = b_dense.reshape(1, NH * H0)

    cparams = pltpu.CompilerParams(dimension_semantics=("parallel",))

    h2d = pl.pallas_call(
        _dense_kernel,
        grid=(8,),
        in_specs=[
            pl.BlockSpec((B // 8, LD), lambda i: (i, 0)),
            pl.BlockSpec((LD, NH * H0), lambda i: (0, 0)),
            pl.BlockSpec((1, NH * H0), lambda i: (0, 0)),
        ],
        out_specs=pl.BlockSpec((B // 8, NH * H0), lambda i: (i, 0)),
        out_shape=jax.ShapeDtypeStruct((B, NH * H0), f32),
        compiler_params=cparams,
    )(x, W_dense, bd)
    hN = h2d.reshape(N, H0)

    wspec = lambda shape: pl.BlockSpec(shape, lambda i: (0, 0))
    y1, ps1, pq1 = pl.pallas_call(
        _l1_kernel,
        grid=(NB,),
        in_specs=[
            pl.BlockSpec((G * NH, H0), lambda i: (i, 0)),
            wspec((H0, H0 * H1)), wspec((1, H0 * H1)), wspec((H0, H1)),
            wspec((1, H1)),
        ],
        out_specs=[
            pl.BlockSpec((G * NH, H1), lambda i: (i, 0)),
            pl.BlockSpec((1, 1, 16), lambda i: (i, 0, 0)),
            pl.BlockSpec((1, 1, 16), lambda i: (i, 0, 0)),
        ],
        out_shape=[
            jax.ShapeDtypeStruct((N, H1), f32),
            jax.ShapeDtypeStruct((NB, 1, 16), f32),
            jax.ShapeDtypeStruct((NB, 1, 16), f32),
        ],
        compiler_params=cparams,
    )(hN, we1, be1, wr1, bs1)

    pspec = pl.BlockSpec((NB, 1, 16), lambda i: (0, 0, 0))
    y2, ps2, pq2 = pl.pallas_call(
        _l2_kernel,
        grid=(NB,),
        in_specs=[
            pl.BlockSpec((G * NH, H1), lambda i: (i, 0)),
            pspec, pspec,
            wspec((1, H1)), wspec((1, H1)),
            wspec((H1, H1 * NF)), wspec((1, H1 * NF)), wspec((H1, 16)),
            wspec((1, 16)),
        ],
        out_specs=[
            pl.BlockSpec((G * NH, 16), lambda i: (i, 0)),
            pl.BlockSpec((1, 1, 16), lambda i: (i, 0, 0)),
            pl.BlockSpec((1, 1, 16), lambda i: (i, 0, 0)),
        ],
        out_shape=[
            jax.ShapeDtypeStruct((N, 16), f32),
            jax.ShapeDtypeStruct((NB, 1, 16), f32),
            jax.ShapeDtypeStruct((NB, 1, 16), f32),
        ],
        compiler_params=cparams,
    )(y1, ps1, pq1, g1, b1, we2, be2, wr2, bs2)

    y3 = pl.pallas_call(
        _bn_kernel,
        grid=(N // RB,),
        in_specs=[
            pl.BlockSpec((RB, 16), lambda i: (i, 0)),
            pspec, pspec,
            wspec((1, 16)), wspec((1, 16)),
        ],
        out_specs=pl.BlockSpec((RB, 16), lambda i: (i, 0)),
        out_shape=jax.ShapeDtypeStruct((N, 16), f32),
        compiler_params=cparams,
    )(y2, ps2, pq2, g2, b2)

    return y3.reshape(B, NH, 16)[:, :, :NF]


# speedup vs baseline: 1.9838x; 1.2987x over previous
import jax
import jax.numpy as jnp
from jax.experimental import pallas as pl
from jax.experimental.pallas import tpu as pltpu

B = 1024
NH = 128
LD = 128
H0 = 16
H1 = 16
NF = 3
ALPHA = 0.2
N = B * NH

G = 8
NB = B // G
RB = 2048

_HI = jax.lax.Precision.HIGHEST
_BF = jnp.bfloat16


def _leaky(x):
    return jnp.where(x >= 0, x, ALPHA * x)


def _dot_bf(a, b):
    return jnp.dot(a.astype(_BF), b.astype(_BF),
                   preferred_element_type=jnp.float32)


def _split3(a):
    p1 = a.astype(_BF).astype(jnp.float32)
    r = a - p1
    p2 = r.astype(_BF).astype(jnp.float32)
    p3 = (r - p2).astype(_BF).astype(jnp.float32)
    return p1, p2, p3


def _sel_dot(sel01, b):
    p1, p2, p3 = _split3(b)
    return _dot_bf(sel01, p1) + _dot_bf(sel01, p2) + _dot_bf(sel01, p3)


def _red_dot(a, red01):
    p1, p2, p3 = _split3(a)
    return _dot_bf(p1, red01) + _dot_bf(p2, red01) + _dot_bf(p3, red01)


def _dense_kernel(x_ref, w_ref, b_ref, o_ref):
    o_ref[...] = _leaky(_dot_bf(x_ref[...], w_ref[...]) + b_ref[...])


def _conv_block(h_blk, we, be, wr, bias, d_out):
    C = 16 * d_out
    hb = h_blk.astype(_BF)
    p1, p2, p3 = _split3(h_blk)
    hs3b = jnp.concatenate([p1, p2, p3], axis=1).astype(_BF)
    iota_n = jax.lax.broadcasted_iota(jnp.int32, (NH, NH), 0)
    iota_m = jax.lax.broadcasted_iota(jnp.int32, (NH, NH), 1)
    diag = jnp.where(iota_n == iota_m, 1e10, 0.0)
    iota_f = iota_m.astype(jnp.float32)
    xs3_parts = []
    for g in range(G):
        sl = slice(g * NH, (g + 1) * NH)
        h_g = h_blk[sl, :]
        h_g_b = hb[sl, :]
        dots = jax.lax.dot_general(h_g_b, h_g_b, (((1,), (1,)), ((), ())),
                                   preferred_element_type=jnp.float32)
        sq = jnp.sum(h_g * h_g, axis=1, keepdims=True)
        d2 = sq + jnp.transpose(sq) - 2.0 * dots + diag
        rowmin = jnp.min(d2, axis=1, keepdims=True)
        cand = jnp.where(d2 <= rowmin, iota_f, jnp.float32(NH))
        nbr = jnp.min(cand, axis=1, keepdims=True)
        sel = jnp.where(cand == nbr, 1.0, 0.0).astype(_BF)
        xs3_parts.append(jnp.dot(sel, hs3b[sl, :],
                                 preferred_element_type=jnp.float32))
    xs3 = jnp.concatenate(xs3_parts, axis=0)
    xs = xs3[:, 0:16] + xs3[:, 16:32] + xs3[:, 32:48]
    ea = xs - h_blk
    theta = _dot_bf(ea, we) + be
    ei = jax.lax.broadcasted_iota(jnp.int32, (16, C), 0)
    ec = jax.lax.broadcasted_iota(jnp.int32, (16, C), 1)
    e1 = jnp.where(ec // d_out == ei, 1.0, 0.0)
    xse = _dot_bf(xs, e1)
    thb = theta.astype(_BF).astype(jnp.float32)
    prod = xse * thb
    ri = jax.lax.broadcasted_iota(jnp.int32, (C, 16), 0)
    rc = jax.lax.broadcasted_iota(jnp.int32, (C, 16), 1)
    red = jnp.where(ri % d_out == rc, 1.0, 0.0)
    msg = _red_dot(prod, red)
    return _dot_bf(h_blk, wr) + msg + bias


def _l1_kernel(h_ref, we_ref, be_ref, wr_ref, b_ref,
               y_ref, ps_ref, pq_ref):
    y = _conv_block(h_ref[...], we_ref[...], be_ref[...], wr_ref[...],
                    b_ref[...], 16)
    y_ref[...] = y
    ps_ref[...] = jnp.sum(y, axis=0, keepdims=True).reshape(1, 1, 16)
    pq_ref[...] = jnp.sum(y * y, axis=0, keepdims=True).reshape(1, 1, 16)


def _l2_kernel(y1_ref, ps1_ref, pq1_ref, g1_ref, b1_ref,
               we_ref, be_ref, wr_ref, bias_ref,
               y2_ref, ps_ref, pq_ref):
    s = jnp.sum(ps1_ref[...], axis=(0, 1))
    q = jnp.sum(pq1_ref[...], axis=(0, 1))
    mean = (s / N).reshape(1, 16)
    var = (q / N).reshape(1, 16) - mean * mean
    denom = jnp.sqrt(var + 1e-5)
    h_blk = _leaky((y1_ref[...] - mean) / denom * g1_ref[...] + b1_ref[...])
    y = _conv_block(h_blk, we_ref[...], be_ref[...], wr_ref[...],
                    bias_ref[...], NF)
    y2_ref[...] = y
    ps_ref[...] = jnp.sum(y, axis=0, keepdims=True).reshape(1, 1, 16)
    pq_ref[...] = jnp.sum(y * y, axis=0, keepdims=True).reshape(1, 1, 16)


def _bn_kernel(y2_ref, ps_ref, pq_ref, g_ref, b_ref, o_ref):
    s = jnp.sum(ps_ref[...], axis=(0, 1))
    q = jnp.sum(pq_ref[...], axis=(0, 1))
    mean = jnp.tile((s / N).reshape(1, 16), (1, 8))
    var = jnp.tile((q / N).reshape(1, 16), (1, 8)) - mean * mean
    denom = jnp.sqrt(var + 1e-5)
    o_ref[...] = _leaky((y2_ref[...] - mean) / denom * g_ref[...] + b_ref[...])


def kernel(x, W_dense, b_dense, W_edge1, b_edge1, W_root1, bias1, bn1_g, bn1_b,
           W_edge2, b_edge2, W_root2, bias2, bn2_g, bn2_b):
    f32 = jnp.float32
    we1 = W_edge1
    be1 = b_edge1.reshape(1, H0 * H1)
    wr1 = W_root1
    bs1 = bias1.reshape(1, H1)
    g1 = bn1_g.reshape(1, H1)
    b1 = bn1_b.reshape(1, H1)
    we2 = W_edge2
    be2 = b_edge2.reshape(1, H1 * NF)
    wr2 = jnp.pad(W_root2, ((0, 0), (0, 16 - NF)))
    bs2 = jnp.pad(bias2.reshape(1, NF), ((0, 0), (0, 16 - NF)))
    g2 = jnp.pad(bn2_g.reshape(1, NF), ((0, 0), (0, 16 - NF)))
    b2 = jnp.pad(bn2_b.reshape(1, NF), ((0, 0), (0, 16 - NF)))
    bd = b_dense.reshape(1, NH * H0)

    cparams = pltpu.CompilerParams(dimension_semantics=("parallel",))

    h2d = pl.pallas_call(
        _dense_kernel,
        grid=(8,),
        in_specs=[
            pl.BlockSpec((B // 8, LD), lambda i: (i, 0)),
            pl.BlockSpec((LD, NH * H0), lambda i: (0, 0)),
            pl.BlockSpec((1, NH * H0), lambda i: (0, 0)),
        ],
        out_specs=pl.BlockSpec((B // 8, NH * H0), lambda i: (i, 0)),
        out_shape=jax.ShapeDtypeStruct((B, NH * H0), f32),
        compiler_params=cparams,
    )(x, W_dense, bd)
    hN = h2d.reshape(N, H0)

    wspec = lambda shape: pl.BlockSpec(shape, lambda i: (0, 0))
    y1, ps1, pq1 = pl.pallas_call(
        _l1_kernel,
        grid=(NB,),
        in_specs=[
            pl.BlockSpec((G * NH, H0), lambda i: (i, 0)),
            wspec((H0, H0 * H1)), wspec((1, H0 * H1)), wspec((H0, H1)),
            wspec((1, H1)),
        ],
        out_specs=[
            pl.BlockSpec((G * NH, H1), lambda i: (i, 0)),
            pl.BlockSpec((1, 1, 16), lambda i: (i, 0, 0)),
            pl.BlockSpec((1, 1, 16), lambda i: (i, 0, 0)),
        ],
        out_shape=[
            jax.ShapeDtypeStruct((N, H1), f32),
            jax.ShapeDtypeStruct((NB, 1, 16), f32),
            jax.ShapeDtypeStruct((NB, 1, 16), f32),
        ],
        compiler_params=cparams,
    )(hN, we1, be1, wr1, bs1)

    pspec = pl.BlockSpec((NB, 1, 16), lambda i: (0, 0, 0))
    y2, ps2, pq2 = pl.pallas_call(
        _l2_kernel,
        grid=(NB,),
        in_specs=[
            pl.BlockSpec((G * NH, H1), lambda i: (i, 0)),
            pspec, pspec,
            wspec((1, H1)), wspec((1, H1)),
            wspec((H1, H1 * NF)), wspec((1, H1 * NF)), wspec((H1, 16)),
            wspec((1, 16)),
        ],
        out_specs=[
            pl.BlockSpec((G * NH, 16), lambda i: (i, 0)),
            pl.BlockSpec((1, 1, 16), lambda i: (i, 0, 0)),
            pl.BlockSpec((1, 1, 16), lambda i: (i, 0, 0)),
        ],
        out_shape=[
            jax.ShapeDtypeStruct((N, 16), f32),
            jax.ShapeDtypeStruct((NB, 1, 16), f32),
            jax.ShapeDtypeStruct((NB, 1, 16), f32),
        ],
        compiler_params=cparams,
    )(y1, ps1, pq1, g1, b1, we2, be2, wr2, bs2)

    y2r = y2.reshape(N // 8, 128)
    g2t = jnp.tile(g2, (1, 8))
    b2t = jnp.tile(b2, (1, 8))
    y3 = pl.pallas_call(
        _bn_kernel,
        grid=((N // 8) // RB,),
        in_specs=[
            pl.BlockSpec((RB, 128), lambda i: (i, 0)),
            pspec, pspec,
            wspec((1, 128)), wspec((1, 128)),
        ],
        out_specs=pl.BlockSpec((RB, 128), lambda i: (i, 0)),
        out_shape=jax.ShapeDtypeStruct((N // 8, 128), f32),
        compiler_params=cparams,
    )(y2r, ps2, pq2, g2t, b2t)

    return y3.reshape(B, NH, 16)[:, :, :NF]


# speedup vs baseline: 2.6774x; 1.3496x over previous
import jax
import jax.numpy as jnp
from jax.experimental import pallas as pl
from jax.experimental.pallas import tpu as pltpu

B = 1024
NH = 128
LD = 128
H0 = 16
H1 = 16
NF = 3
ALPHA = 0.2
N = B * NH

G = 16
NB = B // G
RB = 2048

_HI = jax.lax.Precision.HIGHEST
_BF = jnp.bfloat16


def _leaky(x):
    return jnp.where(x >= 0, x, ALPHA * x)


def _dot_bf(a, b):
    return jnp.dot(a.astype(_BF), b.astype(_BF),
                   preferred_element_type=jnp.float32)


def _split3(a):
    p1 = a.astype(_BF).astype(jnp.float32)
    r = a - p1
    p2 = r.astype(_BF).astype(jnp.float32)
    p3 = (r - p2).astype(_BF).astype(jnp.float32)
    return p1, p2, p3


def _sel_dot(sel01, b):
    p1, p2, p3 = _split3(b)
    return _dot_bf(sel01, p1) + _dot_bf(sel01, p2) + _dot_bf(sel01, p3)


def _red_dot(a, red01):
    p1, p2, p3 = _split3(a)
    return _dot_bf(p1, red01) + _dot_bf(p2, red01) + _dot_bf(p3, red01)


def _dense_kernel(x_ref, w_ref, b_ref, o_ref):
    o_ref[...] = _leaky(_dot_bf(x_ref[...], w_ref[...]) + b_ref[...])


def _conv_block(h_blk, we, be, wr, bias, d_out):
    C = 16 * d_out
    hb = h_blk.astype(_BF)
    p1, p2, p3 = _split3(h_blk)
    hs3b = jnp.concatenate([p1, p2, p3], axis=1).astype(_BF)
    h3 = h_blk.reshape(G, NH, 16)
    hb3 = hb.reshape(G, NH, 16)
    dots = jax.lax.dot_general(hb3, hb3, (((2,), (2,)), ((0,), (0,))),
                               preferred_element_type=jnp.float32)
    sq = jnp.sum(h3 * h3, axis=2, keepdims=True)
    iota_n = jax.lax.broadcasted_iota(jnp.int32, (G, NH, NH), 1)
    iota_m = jax.lax.broadcasted_iota(jnp.int32, (G, NH, NH), 2)
    d2 = sq + jnp.transpose(sq, (0, 2, 1)) - 2.0 * dots \
        + jnp.where(iota_n == iota_m, 1e10, 0.0)
    rowmin = jnp.min(d2, axis=2, keepdims=True)
    iota_f = iota_m.astype(jnp.float32)
    cand = jnp.where(d2 <= rowmin, iota_f, jnp.float32(NH))
    nbr = jnp.min(cand, axis=2, keepdims=True)
    sel = jnp.where(cand == nbr, 1.0, 0.0).astype(_BF)
    xs3 = jax.lax.dot_general(sel, hs3b.reshape(G, NH, 48),
                              (((2,), (1,)), ((0,), (0,))),
                              preferred_element_type=jnp.float32)
    xs3 = xs3.reshape(G * NH, 48)
    xs = xs3[:, 0:16] + xs3[:, 16:32] + xs3[:, 32:48]
    ea = xs - h_blk
    theta = _dot_bf(ea, we) + be
    ei = jax.lax.broadcasted_iota(jnp.int32, (16, C), 0)
    ec = jax.lax.broadcasted_iota(jnp.int32, (16, C), 1)
    e1 = jnp.where(ec // d_out == ei, 1.0, 0.0)
    xse = _dot_bf(xs, e1)
    thb = theta.astype(_BF).astype(jnp.float32)
    prod = xse * thb
    ri = jax.lax.broadcasted_iota(jnp.int32, (C, 16), 0)
    rc = jax.lax.broadcasted_iota(jnp.int32, (C, 16), 1)
    red = jnp.where(ri % d_out == rc, 1.0, 0.0)
    msg = _red_dot(prod, red)
    return _dot_bf(h_blk, wr) + msg + bias


def _l1_kernel(h_ref, we_ref, be_ref, wr_ref, b_ref,
               y_ref, ps_ref, pq_ref):
    y = _conv_block(h_ref[...], we_ref[...], be_ref[...], wr_ref[...],
                    b_ref[...], 16)
    y_ref[...] = y
    ps_ref[...] = jnp.sum(y, axis=0, keepdims=True).reshape(1, 1, 16)
    pq_ref[...] = jnp.sum(y * y, axis=0, keepdims=True).reshape(1, 1, 16)


def _l2_kernel(y1_ref, ps1_ref, pq1_ref, g1_ref, b1_ref,
               we_ref, be_ref, wr_ref, bias_ref,
               y2_ref, ps_ref, pq_ref):
    s = jnp.sum(ps1_ref[...], axis=(0, 1))
    q = jnp.sum(pq1_ref[...], axis=(0, 1))
    mean = (s / N).reshape(1, 16)
    var = (q / N).reshape(1, 16) - mean * mean
    denom = jnp.sqrt(var + 1e-5)
    h_blk = _leaky((y1_ref[...] - mean) / denom * g1_ref[...] + b1_ref[...])
    y = _conv_block(h_blk, we_ref[...], be_ref[...], wr_ref[...],
                    bias_ref[...], NF)
    y2_ref[...] = y
    ps_ref[...] = jnp.sum(y, axis=0, keepdims=True).reshape(1, 1, 16)
    pq_ref[...] = jnp.sum(y * y, axis=0, keepdims=True).reshape(1, 1, 16)


def _bn_kernel(y2_ref, ps_ref, pq_ref, g_ref, b_ref, o_ref):
    s = jnp.sum(ps_ref[...], axis=(0, 1))
    q = jnp.sum(pq_ref[...], axis=(0, 1))
    mean = jnp.tile((s / N).reshape(1, 16), (1, 8))
    var = jnp.tile((q / N).reshape(1, 16), (1, 8)) - mean * mean
    denom = jnp.sqrt(var + 1e-5)
    o_ref[...] = _leaky((y2_ref[...] - mean) / denom * g_ref[...] + b_ref[...])


def kernel(x, W_dense, b_dense, W_edge1, b_edge1, W_root1, bias1, bn1_g, bn1_b,
           W_edge2, b_edge2, W_root2, bias2, bn2_g, bn2_b):
    f32 = jnp.float32
    we1 = W_edge1
    be1 = b_edge1.reshape(1, H0 * H1)
    wr1 = W_root1
    bs1 = bias1.reshape(1, H1)
    g1 = bn1_g.reshape(1, H1)
    b1 = bn1_b.reshape(1, H1)
    we2 = W_edge2
    be2 = b_edge2.reshape(1, H1 * NF)
    wr2 = jnp.pad(W_root2, ((0, 0), (0, 16 - NF)))
    bs2 = jnp.pad(bias2.reshape(1, NF), ((0, 0), (0, 16 - NF)))
    g2 = jnp.pad(bn2_g.reshape(1, NF), ((0, 0), (0, 16 - NF)))
    b2 = jnp.pad(bn2_b.reshape(1, NF), ((0, 0), (0, 16 - NF)))
    bd = b_dense.reshape(1, NH * H0)

    cparams = pltpu.CompilerParams(dimension_semantics=("parallel",))

    h2d = pl.pallas_call(
        _dense_kernel,
        grid=(8,),
        in_specs=[
            pl.BlockSpec((B // 8, LD), lambda i: (i, 0)),
            pl.BlockSpec((LD, NH * H0), lambda i: (0, 0)),
            pl.BlockSpec((1, NH * H0), lambda i: (0, 0)),
        ],
        out_specs=pl.BlockSpec((B // 8, NH * H0), lambda i: (i, 0)),
        out_shape=jax.ShapeDtypeStruct((B, NH * H0), f32),
        compiler_params=cparams,
    )(x, W_dense, bd)
    hN = h2d.reshape(N, H0)

    wspec = lambda shape: pl.BlockSpec(shape, lambda i: (0, 0))
    y1, ps1, pq1 = pl.pallas_call(
        _l1_kernel,
        grid=(NB,),
        in_specs=[
            pl.BlockSpec((G * NH, H0), lambda i: (i, 0)),
            wspec((H0, H0 * H1)), wspec((1, H0 * H1)), wspec((H0, H1)),
            wspec((1, H1)),
        ],
        out_specs=[
            pl.BlockSpec((G * NH, H1), lambda i: (i, 0)),
            pl.BlockSpec((1, 1, 16), lambda i: (i, 0, 0)),
            pl.BlockSpec((1, 1, 16), lambda i: (i, 0, 0)),
        ],
        out_shape=[
            jax.ShapeDtypeStruct((N, H1), f32),
            jax.ShapeDtypeStruct((NB, 1, 16), f32),
            jax.ShapeDtypeStruct((NB, 1, 16), f32),
        ],
        compiler_params=cparams,
    )(hN, we1, be1, wr1, bs1)

    pspec = pl.BlockSpec((NB, 1, 16), lambda i: (0, 0, 0))
    y2, ps2, pq2 = pl.pallas_call(
        _l2_kernel,
        grid=(NB,),
        in_specs=[
            pl.BlockSpec((G * NH, H1), lambda i: (i, 0)),
            pspec, pspec,
            wspec((1, H1)), wspec((1, H1)),
            wspec((H1, H1 * NF)), wspec((1, H1 * NF)), wspec((H1, 16)),
            wspec((1, 16)),
        ],
        out_specs=[
            pl.BlockSpec((G * NH, 16), lambda i: (i, 0)),
            pl.BlockSpec((1, 1, 16), lambda i: (i, 0, 0)),
            pl.BlockSpec((1, 1, 16), lambda i: (i, 0, 0)),
        ],
        out_shape=[
            jax.ShapeDtypeStruct((N, 16), f32),
            jax.ShapeDtypeStruct((NB, 1, 16), f32),
            jax.ShapeDtypeStruct((NB, 1, 16), f32),
        ],
        compiler_params=cparams,
    )(y1, ps1, pq1, g1, b1, we2, be2, wr2, bs2)

    y2r = y2.reshape(N // 8, 128)
    g2t = jnp.tile(g2, (1, 8))
    b2t = jnp.tile(b2, (1, 8))
    y3 = pl.pallas_call(
        _bn_kernel,
        grid=((N // 8) // RB,),
        in_specs=[
            pl.BlockSpec((RB, 128), lambda i: (i, 0)),
            pspec, pspec,
            wspec((1, 128)), wspec((1, 128)),
        ],
        out_specs=pl.BlockSpec((RB, 128), lambda i: (i, 0)),
        out_shape=jax.ShapeDtypeStruct((N // 8, 128), f32),
        compiler_params=cparams,
    )(y2r, ps2, pq2, g2t, b2t)

    return y3.reshape(B, NH, 16)[:, :, :NF]


# speedup vs baseline: 6.2892x; 2.3490x over previous
import jax
import jax.numpy as jnp
from jax.experimental import pallas as pl
from jax.experimental.pallas import tpu as pltpu

B = 1024
NH = 128
LD = 128
H0 = 16
H1 = 16
NF = 3
ALPHA = 0.2
N = B * NH

G = 16
GN = G * NH
NB = B // G
D2P = 8

_BF = jnp.bfloat16


def _leaky(x):
    return jnp.where(x >= 0, x, ALPHA * x)


def _dot_bf(a, b):
    return jnp.dot(a.astype(_BF), b.astype(_BF),
                   preferred_element_type=jnp.float32)


def _split3(a):
    p1 = a.astype(_BF).astype(jnp.float32)
    r = a - p1
    p2 = r.astype(_BF).astype(jnp.float32)
    p3 = (r - p2).astype(_BF).astype(jnp.float32)
    return p1, p2, p3


def _dense_kernel(x_ref, w_ref, b_ref, o_ref):
    o_ref[...] = _leaky(_dot_bf(x_ref[...], w_ref[...]) + b_ref[...])


def _conv_block(hT, weT, beT, wrT, biasT, d_out):
    C = 16 * d_out
    hbT = hT.astype(_BF)
    hb3 = hbT.reshape(16, G, NH).transpose(1, 0, 2)
    dots = jax.lax.dot_general(hb3, hb3, (((1,), (1,)), ((0,), (0,))),
                               preferred_element_type=jnp.float32)
    sqT = jnp.sum(hT * hT, axis=0, keepdims=True)
    sq3 = sqT.reshape(1, G, NH).transpose(1, 0, 2)
    iota_n = jax.lax.broadcasted_iota(jnp.int32, (G, NH, NH), 1)
    iota_m = jax.lax.broadcasted_iota(jnp.int32, (G, NH, NH), 2)
    d2 = sq3 - 2.0 * dots + jnp.where(iota_n == iota_m, 1e10, 0.0)
    rowmin = jnp.min(d2, axis=2, keepdims=True)
    iota_f = iota_m.astype(jnp.float32)
    cand = jnp.where(d2 <= rowmin, iota_f, jnp.float32(NH))
    nbr = jnp.min(cand, axis=2, keepdims=True)
    sel = jnp.where(cand == nbr, 1.0, 0.0).astype(_BF)
    p1, p2, p3 = _split3(hT)
    hs3 = jnp.concatenate([p1, p2, p3], axis=0).astype(_BF)
    hs3 = hs3.reshape(48, G, NH).transpose(1, 0, 2)
    xs3 = jax.lax.dot_general(hs3, sel, (((2,), (2,)), ((0,), (0,))),
                              preferred_element_type=jnp.float32)
    xs3 = xs3.transpose(1, 0, 2).reshape(48, GN)
    xsT = xs3[0:16, :] + xs3[16:32, :] + xs3[32:48, :]
    eaT = xsT - hT
    thetaT = _dot_bf(weT, eaT) + beT
    ec = jax.lax.broadcasted_iota(jnp.int32, (C, 16), 0)
    ei = jax.lax.broadcasted_iota(jnp.int32, (C, 16), 1)
    e1T = jnp.where(ec // d_out == ei, 1.0, 0.0)
    xseT = _dot_bf(e1T, xsT)
    thb = thetaT.astype(_BF).astype(jnp.float32)
    prod = xseT * thb
    msgT = jnp.sum(prod.reshape(16, d_out, GN), axis=0)
    return _dot_bf(wrT, hT) + msgT + biasT


def _psums(yT, d_out):
    ps = jnp.transpose(jnp.sum(yT, axis=1, keepdims=True)).reshape(1, 1, d_out)
    pq = jnp.transpose(jnp.sum(yT * yT, axis=1, keepdims=True)).reshape(1, 1, d_out)
    return ps, pq


def _l1_kernel(h_ref, we_ref, be_ref, wr_ref, b_ref,
               y_ref, ps_ref, pq_ref):
    y = _conv_block(h_ref[...], we_ref[...], be_ref[...], wr_ref[...],
                    b_ref[...], 16)
    y_ref[...] = y
    ps_ref[...], pq_ref[...] = _psums(y, 16)


def _l2_kernel(y1_ref, ps1_ref, pq1_ref, g1_ref, b1_ref,
               we_ref, be_ref, wr_ref, bias_ref,
               y2_ref, ps_ref, pq_ref):
    s = jnp.sum(ps1_ref[...], axis=(0, 1)).reshape(1, 16)
    q = jnp.sum(pq1_ref[...], axis=(0, 1)).reshape(1, 16)
    mean = jnp.transpose(s) / N
    var = jnp.transpose(q) / N - mean * mean
    denom = jnp.sqrt(var + 1e-5)
    hT = _leaky((y1_ref[...] - mean) / denom * g1_ref[...] + b1_ref[...])
    y = _conv_block(hT, we_ref[...], be_ref[...], wr_ref[...],
                    bias_ref[...], D2P)
    y2_ref[...] = y
    ps_ref[...], pq_ref[...] = _psums(y, D2P)


def _bn_kernel(y2_ref, ps_ref, pq_ref, g_ref, b_ref, o_ref):
    s = jnp.sum(ps_ref[...], axis=(0, 1)).reshape(1, D2P)
    q = jnp.sum(pq_ref[...], axis=(0, 1)).reshape(1, D2P)
    mean = jnp.transpose(s) / N
    var = jnp.transpose(q) / N - mean * mean
    denom = jnp.sqrt(var + 1e-5)
    o_ref[...] = _leaky((y2_ref[...] - mean) / denom * g_ref[...] + b_ref[...])


def kernel(x, W_dense, b_dense, W_edge1, b_edge1, W_root1, bias1, bn1_g, bn1_b,
           W_edge2, b_edge2, W_root2, bias2, bn2_g, bn2_b):
    f32 = jnp.float32
    weT1 = jnp.transpose(W_edge1)
    beT1 = b_edge1.reshape(256, 1)
    wrT1 = jnp.transpose(W_root1)
    bsT1 = bias1.reshape(16, 1)
    g1c = bn1_g.reshape(16, 1)
    b1c = bn1_b.reshape(16, 1)
    w2p = jnp.pad(W_edge2.reshape(H1, H1, NF), ((0, 0), (0, 0), (0, D2P - NF)))
    weT2 = jnp.transpose(w2p.reshape(H1, H1 * D2P))
    beT2 = jnp.pad(b_edge2.reshape(H1, NF), ((0, 0), (0, D2P - NF))).reshape(128, 1)
    wrT2 = jnp.pad(jnp.transpose(W_root2), ((0, D2P - NF), (0, 0)))
    bsT2 = jnp.pad(bias2.reshape(NF, 1), ((0, D2P - NF), (0, 0)))
    g2c = jnp.pad(bn2_g.reshape(NF, 1), ((0, D2P - NF), (0, 0)))
    b2c = jnp.pad(bn2_b.reshape(NF, 1), ((0, D2P - NF), (0, 0)))
    bd = b_dense.reshape(1, NH * H0)

    cparams = pltpu.CompilerParams(dimension_semantics=("parallel",))
    wspec = lambda shape: pl.BlockSpec(shape, lambda i: (0, 0))

    h2d = pl.pallas_call(
        _dense_kernel,
        grid=(8,),
        in_specs=[
            pl.BlockSpec((B // 8, LD), lambda i: (i, 0)),
            pl.BlockSpec((LD, NH * H0), lambda i: (0, 0)),
            pl.BlockSpec((1, NH * H0), lambda i: (0, 0)),
        ],
        out_specs=pl.BlockSpec((B // 8, NH * H0), lambda i: (i, 0)),
        out_shape=jax.ShapeDtypeStruct((B, NH * H0), f32),
        compiler_params=cparams,
    )(x, W_dense, bd)
    hT = jnp.transpose(h2d.reshape(N, H0))

    cspec = lambda n: pl.BlockSpec((n, GN), lambda i: (0, i))
    pout = pl.BlockSpec((1, 1, 16), lambda i: (i, 0, 0))
    y1, ps1, pq1 = pl.pallas_call(
        _l1_kernel,
        grid=(NB,),
        in_specs=[
            cspec(16),
            wspec((256, 16)), wspec((256, 1)), wspec((16, 16)), wspec((16, 1)),
        ],
        out_specs=[cspec(16), pout, pout],
        out_shape=[
            jax.ShapeDtypeStruct((16, N), f32),
            jax.ShapeDtypeStruct((NB, 1, 16), f32),
            jax.ShapeDtypeStruct((NB, 1, 16), f32),
        ],
        compiler_params=cparams,
    )(hT, weT1, beT1, wrT1, bsT1)

    pspec = pl.BlockSpec((NB, 1, 16), lambda i: (0, 0, 0))
    pout8 = pl.BlockSpec((1, 1, D2P), lambda i: (i, 0, 0))
    y2, ps2, pq2 = pl.pallas_call(
        _l2_kernel,
        grid=(NB,),
        in_specs=[
            cspec(16), pspec, pspec,
            wspec((16, 1)), wspec((16, 1)),
            wspec((128, 16)), wspec((128, 1)), wspec((D2P, 16)),
            wspec((D2P, 1)),
        ],
        out_specs=[cspec(D2P), pout8, pout8],
        out_shape=[
            jax.ShapeDtypeStruct((D2P, N), f32),
            jax.ShapeDtypeStruct((NB, 1, D2P), f32),
            jax.ShapeDtypeStruct((NB, 1, D2P), f32),
        ],
        compiler_params=cparams,
    )(y1, ps1, pq1, g1c, b1c, weT2, beT2, wrT2, bsT2)

    pspec8 = pl.BlockSpec((NB, 1, D2P), lambda i: (0, 0, 0))
    RB3 = N // 8
    y3 = pl.pallas_call(
        _bn_kernel,
        grid=(8,),
        in_specs=[
            pl.BlockSpec((D2P, RB3), lambda i: (0, i)),
            pspec8, pspec8,
            wspec((D2P, 1)), wspec((D2P, 1)),
        ],
        out_specs=pl.BlockSpec((D2P, RB3), lambda i: (0, i)),
        out_shape=jax.ShapeDtypeStruct((D2P, N), f32),
        compiler_params=cparams,
    )(y2, ps2, pq2, g2c, b2c)

    return jnp.transpose(y3).reshape(B, NH, D2P)[:, :, :NF]


# speedup vs baseline: 7.3277x; 1.1651x over previous
import jax
import jax.numpy as jnp
from jax.experimental import pallas as pl
from jax.experimental.pallas import tpu as pltpu

B = 1024
NH = 128
LD = 128
H0 = 16
H1 = 16
NF = 3
ALPHA = 0.2
N = B * NH

G = 32
GN = G * NH
NB = B // G
D2P = 8

_BF = jnp.bfloat16


def _leaky(x):
    return jnp.where(x >= 0, x, ALPHA * x)


def _dot_bf(a, b):
    return jnp.dot(a.astype(_BF), b.astype(_BF),
                   preferred_element_type=jnp.float32)


def _split3(a):
    p1 = a.astype(_BF).astype(jnp.float32)
    r = a - p1
    p2 = r.astype(_BF).astype(jnp.float32)
    p3 = (r - p2).astype(_BF).astype(jnp.float32)
    return p1, p2, p3


def _dense_kernel(x_ref, w_ref, b_ref, o_ref):
    o_ref[...] = _leaky(_dot_bf(x_ref[...], w_ref[...]) + b_ref[...])


def _conv_block(hT, weT, beT, wrT, biasT, d_out):
    C = 16 * d_out
    hbT = hT.astype(_BF)
    hb3 = hbT.reshape(16, G, NH).transpose(1, 0, 2)
    dots = jax.lax.dot_general(hb3, hb3, (((1,), (1,)), ((0,), (0,))),
                               preferred_element_type=jnp.float32)
    sqT = jnp.sum(hT * hT, axis=0, keepdims=True)
    sq3 = sqT.reshape(1, G, NH).transpose(1, 0, 2)
    iota_n = jax.lax.broadcasted_iota(jnp.int32, (G, NH, NH), 1)
    iota_m = jax.lax.broadcasted_iota(jnp.int32, (G, NH, NH), 2)
    d2 = sq3 - 2.0 * dots + jnp.where(iota_n == iota_m, 1e10, 0.0)
    rowmin = jnp.min(d2, axis=2, keepdims=True)
    iota_f = iota_m.astype(jnp.float32)
    cand = jnp.where(d2 <= rowmin, iota_f, jnp.float32(NH))
    nbr = jnp.min(cand, axis=2, keepdims=True)
    sel = jnp.where(cand == nbr, 1.0, 0.0).astype(_BF)
    p1, p2, p3 = _split3(hT)
    hs3 = jnp.concatenate([p1, p2, p3], axis=0).astype(_BF)
    hs3 = hs3.reshape(48, G, NH).transpose(1, 0, 2)
    xs3 = jax.lax.dot_general(hs3, sel, (((2,), (2,)), ((0,), (0,))),
                              preferred_element_type=jnp.float32)
    xs3 = xs3.transpose(1, 0, 2).reshape(48, GN)
    xsT = xs3[0:16, :] + xs3[16:32, :] + xs3[32:48, :]
    eaT = xsT - hT
    thetaT = _dot_bf(weT, eaT) + beT
    ec = jax.lax.broadcasted_iota(jnp.int32, (C, 16), 0)
    ei = jax.lax.broadcasted_iota(jnp.int32, (C, 16), 1)
    e1T = jnp.where(ec // d_out == ei, 1.0, 0.0)
    xseT = _dot_bf(e1T, xsT)
    thb = thetaT.astype(_BF).astype(jnp.float32)
    prod = xseT * thb
    msgT = jnp.sum(prod.reshape(16, d_out, GN), axis=0)
    return _dot_bf(wrT, hT) + msgT + biasT


def _psums(yT, d_out):
    ps = jnp.transpose(jnp.sum(yT, axis=1, keepdims=True)).reshape(1, 1, d_out)
    pq = jnp.transpose(jnp.sum(yT * yT, axis=1, keepdims=True)).reshape(1, 1, d_out)
    return ps, pq


def _l1_kernel(h_ref, we_ref, be_ref, wr_ref, b_ref,
               y_ref, ps_ref, pq_ref):
    y = _conv_block(h_ref[...], we_ref[...], be_ref[...], wr_ref[...],
                    b_ref[...], 16)
    y_ref[...] = y
    ps_ref[...], pq_ref[...] = _psums(y, 16)


def _l2_kernel(y1_ref, ps1_ref, pq1_ref, g1_ref, b1_ref,
               we_ref, be_ref, wr_ref, bias_ref,
               y2_ref, ps_ref, pq_ref):
    s = jnp.sum(ps1_ref[...], axis=(0, 1)).reshape(1, 16)
    q = jnp.sum(pq1_ref[...], axis=(0, 1)).reshape(1, 16)
    mean = jnp.transpose(s) / N
    var = jnp.transpose(q) / N - mean * mean
    denom = jnp.sqrt(var + 1e-5)
    hT = _leaky((y1_ref[...] - mean) / denom * g1_ref[...] + b1_ref[...])
    y = _conv_block(hT, we_ref[...], be_ref[...], wr_ref[...],
                    bias_ref[...], D2P)
    y2_ref[...] = y
    ps_ref[...], pq_ref[...] = _psums(y, D2P)


def _bn_kernel(y2_ref, ps_ref, pq_ref, g_ref, b_ref, o_ref):
    s = jnp.sum(ps_ref[...], axis=(0, 1)).reshape(1, D2P)
    q = jnp.sum(pq_ref[...], axis=(0, 1)).reshape(1, D2P)
    mean = jnp.transpose(s) / N
    var = jnp.transpose(q) / N - mean * mean
    denom = jnp.sqrt(var + 1e-5)
    o_ref[...] = _leaky((y2_ref[...] - mean) / denom * g_ref[...] + b_ref[...])


def kernel(x, W_dense, b_dense, W_edge1, b_edge1, W_root1, bias1, bn1_g, bn1_b,
           W_edge2, b_edge2, W_root2, bias2, bn2_g, bn2_b):
    f32 = jnp.float32
    weT1 = jnp.transpose(W_edge1)
    beT1 = b_edge1.reshape(256, 1)
    wrT1 = jnp.transpose(W_root1)
    bsT1 = bias1.reshape(16, 1)
    g1c = bn1_g.reshape(16, 1)
    b1c = bn1_b.reshape(16, 1)
    w2p = jnp.pad(W_edge2.reshape(H1, H1, NF), ((0, 0), (0, 0), (0, D2P - NF)))
    weT2 = jnp.transpose(w2p.reshape(H1, H1 * D2P))
    beT2 = jnp.pad(b_edge2.reshape(H1, NF), ((0, 0), (0, D2P - NF))).reshape(128, 1)
    wrT2 = jnp.pad(jnp.transpose(W_root2), ((0, D2P - NF), (0, 0)))
    bsT2 = jnp.pad(bias2.reshape(NF, 1), ((0, D2P - NF), (0, 0)))
    g2c = jnp.pad(bn2_g.reshape(NF, 1), ((0, D2P - NF), (0, 0)))
    b2c = jnp.pad(bn2_b.reshape(NF, 1), ((0, D2P - NF), (0, 0)))
    bd = b_dense.reshape(1, NH * H0)

    cparams = pltpu.CompilerParams(dimension_semantics=("parallel",))
    wspec = lambda shape: pl.BlockSpec(shape, lambda i: (0, 0))

    h2d = pl.pallas_call(
        _dense_kernel,
        grid=(8,),
        in_specs=[
            pl.BlockSpec((B // 8, LD), lambda i: (i, 0)),
            pl.BlockSpec((LD, NH * H0), lambda i: (0, 0)),
            pl.BlockSpec((1, NH * H0), lambda i: (0, 0)),
        ],
        out_specs=pl.BlockSpec((B // 8, NH * H0), lambda i: (i, 0)),
        out_shape=jax.ShapeDtypeStruct((B, NH * H0), f32),
        compiler_params=cparams,
    )(x, W_dense, bd)
    hT = jnp.transpose(h2d.reshape(N, H0))

    cspec = lambda n: pl.BlockSpec((n, GN), lambda i: (0, i))
    pout = pl.BlockSpec((1, 1, 16), lambda i: (i, 0, 0))
    y1, ps1, pq1 = pl.pallas_call(
        _l1_kernel,
        grid=(NB,),
        in_specs=[
            cspec(16),
            wspec((256, 16)), wspec((256, 1)), wspec((16, 16)), wspec((16, 1)),
        ],
        out_specs=[cspec(16), pout, pout],
        out_shape=[
            jax.ShapeDtypeStruct((16, N), f32),
            jax.ShapeDtypeStruct((NB, 1, 16), f32),
            jax.ShapeDtypeStruct((NB, 1, 16), f32),
        ],
        compiler_params=cparams,
    )(hT, weT1, beT1, wrT1, bsT1)

    pspec = pl.BlockSpec((NB, 1, 16), lambda i: (0, 0, 0))
    pout8 = pl.BlockSpec((1, 1, D2P), lambda i: (i, 0, 0))
    y2, ps2, pq2 = pl.pallas_call(
        _l2_kernel,
        grid=(NB,),
        in_specs=[
            cspec(16), pspec, pspec,
            wspec((16, 1)), wspec((16, 1)),
            wspec((128, 16)), wspec((128, 1)), wspec((D2P, 16)),
            wspec((D2P, 1)),
        ],
        out_specs=[cspec(D2P), pout8, pout8],
        out_shape=[
            jax.ShapeDtypeStruct((D2P, N), f32),
            jax.ShapeDtypeStruct((NB, 1, D2P), f32),
            jax.ShapeDtypeStruct((NB, 1, D2P), f32),
        ],
        compiler_params=cparams,
    )(y1, ps1, pq1, g1c, b1c, weT2, beT2, wrT2, bsT2)

    pspec8 = pl.BlockSpec((NB, 1, D2P), lambda i: (0, 0, 0))
    RB3 = N // 8
    y3 = pl.pallas_call(
        _bn_kernel,
        grid=(8,),
        in_specs=[
            pl.BlockSpec((D2P, RB3), lambda i: (0, i)),
            pspec8, pspec8,
            wspec((D2P, 1)), wspec((D2P, 1)),
        ],
        out_specs=pl.BlockSpec((D2P, RB3), lambda i: (0, i)),
        out_shape=jax.ShapeDtypeStruct((D2P, N), f32),
        compiler_params=cparams,
    )(y2, ps2, pq2, g2c, b2c)

    return jnp.transpose(y3).reshape(B, NH, D2P)[:, :, :NF]


# speedup vs baseline: 7.8521x; 1.0716x over previous
import jax
import jax.numpy as jnp
from jax.experimental import pallas as pl
from jax.experimental.pallas import tpu as pltpu

B = 1024
NH = 128
LD = 128
H0 = 16
H1 = 16
NF = 3
ALPHA = 0.2
N = B * NH

G = 64
GN = G * NH
NB = B // G
D2P = 8

_BF = jnp.bfloat16


def _leaky(x):
    return jnp.where(x >= 0, x, ALPHA * x)


def _dot_bf(a, b):
    return jnp.dot(a.astype(_BF), b.astype(_BF),
                   preferred_element_type=jnp.float32)


def _split3(a):
    p1 = a.astype(_BF).astype(jnp.float32)
    r = a - p1
    p2 = r.astype(_BF).astype(jnp.float32)
    p3 = (r - p2).astype(_BF).astype(jnp.float32)
    return p1, p2, p3


def _dense_kernel(x_ref, w_ref, b_ref, o_ref):
    o_ref[...] = _leaky(_dot_bf(x_ref[...], w_ref[...]) + b_ref[...])


def _conv_block(hT, weT, beT, wrT, biasT, d_out):
    C = 16 * d_out
    hbT = hT.astype(_BF)
    hb3 = hbT.reshape(16, G, NH).transpose(1, 0, 2)
    dots = jax.lax.dot_general(hb3, hb3, (((1,), (1,)), ((0,), (0,))),
                               preferred_element_type=jnp.float32)
    sqT = jnp.sum(hT * hT, axis=0, keepdims=True)
    sq3 = sqT.reshape(1, G, NH).transpose(1, 0, 2)
    iota_n = jax.lax.broadcasted_iota(jnp.int32, (G, NH, NH), 1)
    iota_m = jax.lax.broadcasted_iota(jnp.int32, (G, NH, NH), 2)
    d2 = sq3 - 2.0 * dots + jnp.where(iota_n == iota_m, 1e10, 0.0)
    rowmin = jnp.min(d2, axis=2, keepdims=True)
    iota_f = iota_m.astype(jnp.float32)
    cand = jnp.where(d2 <= rowmin, iota_f, jnp.float32(NH))
    nbr = jnp.min(cand, axis=2, keepdims=True)
    sel = jnp.where(cand == nbr, 1.0, 0.0).astype(_BF)
    p1, p2, p3 = _split3(hT)
    hs3 = jnp.concatenate([p1, p2, p3], axis=0).astype(_BF)
    hs3 = hs3.reshape(48, G, NH).transpose(1, 0, 2)
    xs3 = jax.lax.dot_general(hs3, sel, (((2,), (2,)), ((0,), (0,))),
                              preferred_element_type=jnp.float32)
    xs3 = xs3.transpose(1, 0, 2).reshape(48, GN)
    xsT = xs3[0:16, :] + xs3[16:32, :] + xs3[32:48, :]
    eaT = xsT - hT
    thetaT = _dot_bf(weT, eaT) + beT
    ec = jax.lax.broadcasted_iota(jnp.int32, (C, 16), 0)
    ei = jax.lax.broadcasted_iota(jnp.int32, (C, 16), 1)
    e1T = jnp.where(ec // d_out == ei, 1.0, 0.0)
    xseT = _dot_bf(e1T, xsT)
    thb = thetaT.astype(_BF).astype(jnp.float32)
    prod = xseT * thb
    msgT = jnp.sum(prod.reshape(16, d_out, GN), axis=0)
    return _dot_bf(wrT, hT) + msgT + biasT


def _psums(yT, d_out):
    ps = jnp.transpose(jnp.sum(yT, axis=1, keepdims=True)).reshape(1, 1, d_out)
    pq = jnp.transpose(jnp.sum(yT * yT, axis=1, keepdims=True)).reshape(1, 1, d_out)
    return ps, pq


def _l1_kernel(h_ref, we_ref, be_ref, wr_ref, b_ref,
               y_ref, ps_ref, pq_ref):
    y = _conv_block(h_ref[...], we_ref[...], be_ref[...], wr_ref[...],
                    b_ref[...], 16)
    y_ref[...] = y
    ps_ref[...], pq_ref[...] = _psums(y, 16)


def _l2_kernel(y1_ref, ps1_ref, pq1_ref, g1_ref, b1_ref,
               we_ref, be_ref, wr_ref, bias_ref,
               y2_ref, ps_ref, pq_ref):
    s = jnp.sum(ps1_ref[...], axis=(0, 1)).reshape(1, 16)
    q = jnp.sum(pq1_ref[...], axis=(0, 1)).reshape(1, 16)
    mean = jnp.transpose(s) / N
    var = jnp.transpose(q) / N - mean * mean
    denom = jnp.sqrt(var + 1e-5)
    hT = _leaky((y1_ref[...] - mean) / denom * g1_ref[...] + b1_ref[...])
    y = _conv_block(hT, we_ref[...], be_ref[...], wr_ref[...],
                    bias_ref[...], D2P)
    y2_ref[...] = y
    ps_ref[...], pq_ref[...] = _psums(y, D2P)


def _bn_kernel(y2_ref, ps_ref, pq_ref, g_ref, b_ref, o_ref):
    s = jnp.sum(ps_ref[...], axis=(0, 1)).reshape(1, D2P)
    q = jnp.sum(pq_ref[...], axis=(0, 1)).reshape(1, D2P)
    mean = jnp.transpose(s) / N
    var = jnp.transpose(q) / N - mean * mean
    denom = jnp.sqrt(var + 1e-5)
    o_ref[...] = _leaky((y2_ref[...] - mean) / denom * g_ref[...] + b_ref[...])


def kernel(x, W_dense, b_dense, W_edge1, b_edge1, W_root1, bias1, bn1_g, bn1_b,
           W_edge2, b_edge2, W_root2, bias2, bn2_g, bn2_b):
    f32 = jnp.float32
    weT1 = jnp.transpose(W_edge1)
    beT1 = b_edge1.reshape(256, 1)
    wrT1 = jnp.transpose(W_root1)
    bsT1 = bias1.reshape(16, 1)
    g1c = bn1_g.reshape(16, 1)
    b1c = bn1_b.reshape(16, 1)
    w2p = jnp.pad(W_edge2.reshape(H1, H1, NF), ((0, 0), (0, 0), (0, D2P - NF)))
    weT2 = jnp.transpose(w2p.reshape(H1, H1 * D2P))
    beT2 = jnp.pad(b_edge2.reshape(H1, NF), ((0, 0), (0, D2P - NF))).reshape(128, 1)
    wrT2 = jnp.pad(jnp.transpose(W_root2), ((0, D2P - NF), (0, 0)))
    bsT2 = jnp.pad(bias2.reshape(NF, 1), ((0, D2P - NF), (0, 0)))
    g2c = jnp.pad(bn2_g.reshape(NF, 1), ((0, D2P - NF), (0, 0)))
    b2c = jnp.pad(bn2_b.reshape(NF, 1), ((0, D2P - NF), (0, 0)))
    bd = b_dense.reshape(1, NH * H0)

    cparams = pltpu.CompilerParams(dimension_semantics=("parallel",))
    wspec = lambda shape: pl.BlockSpec(shape, lambda i: (0, 0))

    h2d = pl.pallas_call(
        _dense_kernel,
        grid=(8,),
        in_specs=[
            pl.BlockSpec((B // 8, LD), lambda i: (i, 0)),
            pl.BlockSpec((LD, NH * H0), lambda i: (0, 0)),
            pl.BlockSpec((1, NH * H0), lambda i: (0, 0)),
        ],
        out_specs=pl.BlockSpec((B // 8, NH * H0), lambda i: (i, 0)),
        out_shape=jax.ShapeDtypeStruct((B, NH * H0), f32),
        compiler_params=cparams,
    )(x, W_dense, bd)
    hT = jnp.transpose(h2d.reshape(N, H0))

    cspec = lambda n: pl.BlockSpec((n, GN), lambda i: (0, i))
    pout = pl.BlockSpec((1, 1, 16), lambda i: (i, 0, 0))
    y1, ps1, pq1 = pl.pallas_call(
        _l1_kernel,
        grid=(NB,),
        in_specs=[
            cspec(16),
            wspec((256, 16)), wspec((256, 1)), wspec((16, 16)), wspec((16, 1)),
        ],
        out_specs=[cspec(16), pout, pout],
        out_shape=[
            jax.ShapeDtypeStruct((16, N), f32),
            jax.ShapeDtypeStruct((NB, 1, 16), f32),
            jax.ShapeDtypeStruct((NB, 1, 16), f32),
        ],
        compiler_params=cparams,
    )(hT, weT1, beT1, wrT1, bsT1)

    pspec = pl.BlockSpec((NB, 1, 16), lambda i: (0, 0, 0))
    pout8 = pl.BlockSpec((1, 1, D2P), lambda i: (i, 0, 0))
    y2, ps2, pq2 = pl.pallas_call(
        _l2_kernel,
        grid=(NB,),
        in_specs=[
            cspec(16), pspec, pspec,
            wspec((16, 1)), wspec((16, 1)),
            wspec((128, 16)), wspec((128, 1)), wspec((D2P, 16)),
            wspec((D2P, 1)),
        ],
        out_specs=[cspec(D2P), pout8, pout8],
        out_shape=[
            jax.ShapeDtypeStruct((D2P, N), f32),
            jax.ShapeDtypeStruct((NB, 1, D2P), f32),
            jax.ShapeDtypeStruct((NB, 1, D2P), f32),
        ],
        compiler_params=cparams,
    )(y1, ps1, pq1, g1c, b1c, weT2, beT2, wrT2, bsT2)

    pspec8 = pl.BlockSpec((NB, 1, D2P), lambda i: (0, 0, 0))
    RB3 = N // 8
    y3 = pl.pallas_call(
        _bn_kernel,
        grid=(8,),
        in_specs=[
            pl.BlockSpec((D2P, RB3), lambda i: (0, i)),
            pspec8, pspec8,
            wspec((D2P, 1)), wspec((D2P, 1)),
        ],
        out_specs=pl.BlockSpec((D2P, RB3), lambda i: (0, i)),
        out_shape=jax.ShapeDtypeStruct((D2P, N), f32),
        compiler_params=cparams,
    )(y2, ps2, pq2, g2c, b2c)

    return jnp.transpose(y3).reshape(B, NH, D2P)[:, :, :NF]
